# baseline (speedup 1.0000x reference)
import functools

import numpy as np
import jax
import jax.numpy as jnp
from jax import lax
from jax.experimental import pallas as pl
from jax.experimental.pallas import tpu as pltpu

NUM_CLASSES = 133
NUM_TOTAL = NUM_CLASSES + 1
Q = 200
TOPK = 100
IN_HW = 128
OUT_HW = 512
MASKS_PER_STEP = 10


_BIG = 2**30


def _topk_kernel(x_ref, u134_ref, l200_ref,
                 vals3_ref, lab_ref, qid_ref, gs_ref, sv_ref):
    x = x_ref[...]
    m = jnp.max(x, axis=1, keepdims=True)
    e = jnp.exp(x - m)
    s = e / jnp.sum(e, axis=1, keepdims=True)
    col = lax.broadcasted_iota(jnp.int32, (Q, NUM_TOTAL), 1)
    valid = col < NUM_CLASSES
    sv = jnp.where(valid, s, 0.0)
    bits = jnp.where(valid, lax.bitcast_convert_type(s, jnp.int32), 0)
    row = lax.broadcasted_iota(jnp.int32, (Q, NUM_TOTAL), 0)
    fi = row * NUM_CLASSES + col

    def vstep(_, lohi):
        lo, hi = lohi
        mid = (lo + hi) // 2
        g = jnp.sum(jnp.sum((bits > mid).astype(jnp.float32), axis=1,
                            keepdims=True), axis=0, keepdims=True)[0, 0]
        big = g >= TOPK
        return jnp.where(big, mid, lo), jnp.where(big, hi, mid)

    lo0 = jnp.int32(-1)
    hi0 = jnp.int32(0x3F800001)
    _, v100 = lax.fori_loop(0, 31, vstep, (lo0, hi0))
    eqv = bits == v100
    n1 = jnp.sum(jnp.sum((bits > v100).astype(jnp.float32), axis=1,
                         keepdims=True), axis=0, keepdims=True)[0, 0]
    needed = jnp.int32(TOPK) - n1.astype(jnp.int32)

    def fstep(_, lohi):
        lo, hi = lohi
        mid = (lo + hi) // 2
        g2 = jnp.sum(jnp.sum(jnp.where(eqv & (fi <= mid), 1.0, 0.0), axis=1,
                             keepdims=True), axis=0, keepdims=True)[0, 0]
        small = g2.astype(jnp.int32) < needed
        return jnp.where(small, mid, lo), jnp.where(small, hi, mid)

    _, mstar = lax.fori_loop(0, 16, fstep, (jnp.int32(-1),
                                            jnp.int32(Q * NUM_CLASSES)))
    sel = (bits > v100) | (eqv & (fi <= mstar))

    selb = sel.astype(jnp.bfloat16)
    rowcum = jnp.dot(selb, u134_ref[...], preferred_element_type=jnp.float32)
    rowtot = rowcum[:, NUM_TOTAL - 1:NUM_TOTAL]
    offs = jnp.dot(l200_ref[...], rowtot.astype(jnp.bfloat16),
                   preferred_element_type=jnp.float32)
    gslot = jnp.where(sel, rowcum + offs - 1.0, -1.0)
    gs_ref[...] = gslot
    sv_ref[...] = sv

    kio = lax.broadcasted_iota(jnp.int32, (1, 128), 1).astype(jnp.float32)
    colf = lax.broadcasted_iota(jnp.int32, (1, NUM_TOTAL), 1).astype(jnp.float32)

    def crow(q, acc):
        vacc, facc = acc
        grow = gs_ref[pl.ds(q, 1), :]
        gcol = jnp.transpose(grow)
        hot = (gcol == kio).astype(jnp.bfloat16)
        vrow = sv_ref[pl.ds(q, 1), :]
        vh = vrow.astype(jnp.bfloat16)
        r1 = vrow - vh.astype(jnp.float32)
        vm = r1.astype(jnp.bfloat16)
        vl = (r1 - vm.astype(jnp.float32)).astype(jnp.bfloat16)
        firow = q.astype(jnp.float32) * NUM_CLASSES + colf
        fh = firow.astype(jnp.bfloat16)
        fl = (firow - fh.astype(jnp.float32)).astype(jnp.bfloat16)
        vacc = vacc + jnp.dot(vh, hot, preferred_element_type=jnp.float32) \
            + jnp.dot(vm, hot, preferred_element_type=jnp.float32) \
            + jnp.dot(vl, hot, preferred_element_type=jnp.float32)
        facc = facc + jnp.dot(fh, hot, preferred_element_type=jnp.float32) \
            + jnp.dot(fl, hot, preferred_element_type=jnp.float32)
        return vacc, facc

    z128 = jnp.zeros((1, 128), jnp.float32)
    selv, selfi = lax.fori_loop(0, Q, crow, (z128, z128))

    vrow_b = jnp.broadcast_to(selv, (128, 128))
    firow_b = jnp.broadcast_to(selfi, (128, 128))
    vcol_b = jnp.broadcast_to(jnp.transpose(selv), (128, 128))
    ficol_b = jnp.broadcast_to(jnp.transpose(selfi), (128, 128))
    gt = (vrow_b > vcol_b) | ((vrow_b == vcol_b) & (firow_b < ficol_b))
    rankcol = jnp.sum(gt.astype(jnp.float32), axis=1, keepdims=True)
    krow = lax.broadcasted_iota(jnp.int32, (1, 128), 1).astype(jnp.float32)
    hot = (rankcol == krow).astype(jnp.bfloat16)
    svh = selv.astype(jnp.bfloat16)
    sr1 = selv - svh.astype(jnp.float32)
    svm = sr1.astype(jnp.bfloat16)
    svl = (sr1 - svm.astype(jnp.float32)).astype(jnp.bfloat16)
    out_v = (jnp.dot(svh, hot, preferred_element_type=jnp.float32)
             + jnp.dot(svm, hot, preferred_element_type=jnp.float32)
             + jnp.dot(svl, hot, preferred_element_type=jnp.float32))
    sfh = selfi.astype(jnp.bfloat16)
    sfl = (selfi - sfh.astype(jnp.float32)).astype(jnp.bfloat16)
    out_f = (jnp.dot(sfh, hot, preferred_element_type=jnp.float32)
             + jnp.dot(sfl, hot, preferred_element_type=jnp.float32))
    fi_i = out_f.astype(jnp.int32)
    lab_ref[...] = fi_i % NUM_CLASSES
    qid_ref[...] = fi_i // NUM_CLASSES
    ovm_mat = jnp.broadcast_to(jnp.transpose(out_v), (128, 128))
    vals3_ref[...] = ovm_mat.reshape(128, 1, 128)


def _run_topk(mask_cls2d):
    u134 = jnp.asarray(np.triu(np.ones((NUM_TOTAL, NUM_TOTAL))),
                       dtype=jnp.bfloat16)
    l200 = jnp.asarray(np.tril(np.ones((Q, Q)), k=-1), dtype=jnp.bfloat16)
    vals3, lab, qid = pl.pallas_call(
        _topk_kernel,
        out_shape=(
            jax.ShapeDtypeStruct((128, 1, 128), jnp.float32),
            jax.ShapeDtypeStruct((1, 128), jnp.int32),
            jax.ShapeDtypeStruct((1, 128), jnp.int32),
        ),
        scratch_shapes=[pltpu.VMEM((Q, NUM_TOTAL), jnp.float32),
                        pltpu.VMEM((Q, NUM_TOTAL), jnp.float32)],
    )(mask_cls2d, u134, l200)
    return vals3, lab[0, :TOPK], qid[0, :TOPK]


def _split3_bf16(a):
    ah = a.astype(jnp.bfloat16)
    r1 = a - ah.astype(jnp.float32)
    am = r1.astype(jnp.bfloat16)
    al = (r1 - am.astype(jnp.float32)).astype(jnp.bfloat16)
    return ah, am, al


def _mask_kernel(qid_ref, *refs):
    xrefs = refs[:MASKS_PER_STEP]
    u_ref, ut_ref, tv_ref, pred_ref, score_ref = refs[MASKS_PER_STEP:]
    u = u_ref[...]
    ut = ut_ref[...]
    for j, xr in enumerate(xrefs):
        x = xr[0]
        xh, xm, xl = _split3_bf16(x)
        y = (jnp.dot(u, xh, preferred_element_type=jnp.float32)
             + jnp.dot(u, xm, preferred_element_type=jnp.float32)
             + jnp.dot(u, xl, preferred_element_type=jnp.float32))
        yh, ym, yl = _split3_bf16(y)
        z = (jnp.dot(yh, ut, preferred_element_type=jnp.float32)
             + jnp.dot(ym, ut, preferred_element_type=jnp.float32)
             + jnp.dot(yl, ut, preferred_element_type=jnp.float32))
        sig = jax.nn.sigmoid(z)
        pos = z > 0.0
        pred_ref[j] = pos
        posf = pos.astype(jnp.float32)
        num = jnp.sum(sig * posf)
        den = jnp.sum(posf)
        score_ref[j, 0, :] = tv_ref[j, 0, :] * num / (den + 1e-6)


def _bilinear_matrix(out_size, in_size):
    sample = (np.arange(out_size) + 0.5) * (in_size / out_size) - 0.5
    w = np.maximum(0.0, 1.0 - np.abs(sample[:, None] - np.arange(in_size)[None, :]))
    w = w / w.sum(axis=1, keepdims=True)
    return w.astype(np.float32)


def _run_masks(mask_pred3d, qid, tv):
    u_np = _bilinear_matrix(OUT_HW, IN_HW)
    u = jnp.asarray(u_np, dtype=jnp.bfloat16)
    ut = jnp.asarray(u_np.T, dtype=jnp.bfloat16)
    grid_spec = pltpu.PrefetchScalarGridSpec(
        num_scalar_prefetch=1,
        grid=(TOPK // MASKS_PER_STEP,),
        in_specs=(
            [pl.BlockSpec(
                (1, IN_HW, IN_HW),
                functools.partial(
                    lambda jj, i, qid_ref: (qid_ref[MASKS_PER_STEP * i + jj],
                                            0, 0), j))
             for j in range(MASKS_PER_STEP)] +
            [pl.BlockSpec((OUT_HW, IN_HW), lambda i, qid_ref: (0, 0)),
             pl.BlockSpec((IN_HW, OUT_HW), lambda i, qid_ref: (0, 0)),
             pl.BlockSpec((MASKS_PER_STEP, 1, 128),
                          lambda i, qid_ref: (i, 0, 0))]
        ),
        out_specs=[
            pl.BlockSpec((MASKS_PER_STEP, OUT_HW, OUT_HW),
                         lambda i, qid_ref: (i, 0, 0)),
            pl.BlockSpec((MASKS_PER_STEP, 1, 128),
                         lambda i, qid_ref: (i, 0, 0)),
        ],
    )
    pred, score = pl.pallas_call(
        _mask_kernel,
        grid_spec=grid_spec,
        out_shape=(
            jax.ShapeDtypeStruct((TOPK, OUT_HW, OUT_HW), jnp.bool_),
            jax.ShapeDtypeStruct((TOPK, 1, 128), jnp.float32),
        ),
    )(qid, *([mask_pred3d] * MASKS_PER_STEP), u, ut, tv)
    return pred, score[:, 0, 0]


def kernel(mask_cls, mask_pred):
    mask_cls2d = mask_cls.reshape(Q, NUM_TOTAL)
    vals3, labels, qid = _run_topk(mask_cls2d)
    pred_masks, final_scores = _run_masks(mask_pred[0], qid, vals3)
    return final_scores, labels, pred_masks


# speedup vs baseline: 1.3777x; 1.3777x over previous
import functools

import numpy as np
import jax
import jax.numpy as jnp
from jax import lax
from jax.experimental import pallas as pl
from jax.experimental.pallas import tpu as pltpu

NUM_CLASSES = 133
NUM_TOTAL = NUM_CLASSES + 1
Q = 200
TOPK = 100
IN_HW = 128
OUT_HW = 512
MASKS_PER_STEP = 10


_BIG = 2**30


def _topk_kernel(x_ref, u134_ref, l200_ref,
                 vals3_ref, lab_ref, qid_ref):
    x = x_ref[...]
    m = jnp.max(x, axis=1, keepdims=True)
    e = jnp.exp(x - m)
    s = e / jnp.sum(e, axis=1, keepdims=True)
    col = lax.broadcasted_iota(jnp.int32, (Q, NUM_TOTAL), 1)
    valid = col < NUM_CLASSES
    sv = jnp.where(valid, s, 0.0)
    bits = jnp.where(valid, lax.bitcast_convert_type(s, jnp.int32), 0)
    row = lax.broadcasted_iota(jnp.int32, (Q, NUM_TOTAL), 0)
    fi = row * NUM_CLASSES + col

    def vstep(_, lohi):
        lo, hi = lohi
        mid = (lo + hi) // 2
        g = jnp.sum(jnp.sum((bits > mid).astype(jnp.float32), axis=1,
                            keepdims=True), axis=0, keepdims=True)[0, 0]
        big = g >= TOPK
        return jnp.where(big, mid, lo), jnp.where(big, hi, mid)

    lo0 = jnp.int32(-1)
    hi0 = jnp.int32(0x3F800001)
    _, v100 = lax.fori_loop(0, 31, vstep, (lo0, hi0))
    eqv = bits == v100
    n1 = jnp.sum(jnp.sum((bits > v100).astype(jnp.float32), axis=1,
                         keepdims=True), axis=0, keepdims=True)[0, 0]
    needed = jnp.int32(TOPK) - n1.astype(jnp.int32)

    def fstep(_, lohi):
        lo, hi = lohi
        mid = (lo + hi) // 2
        g2 = jnp.sum(jnp.sum(jnp.where(eqv & (fi <= mid), 1.0, 0.0), axis=1,
                             keepdims=True), axis=0, keepdims=True)[0, 0]
        small = g2.astype(jnp.int32) < needed
        return jnp.where(small, mid, lo), jnp.where(small, hi, mid)

    _, mstar = lax.fori_loop(0, 16, fstep, (jnp.int32(-1),
                                            jnp.int32(Q * NUM_CLASSES)))
    sel = (bits > v100) | (eqv & (fi <= mstar))

    selb = sel.astype(jnp.bfloat16)
    rowcum = jnp.dot(selb, u134_ref[...], preferred_element_type=jnp.float32)
    rowtot = rowcum[:, NUM_TOTAL - 1:NUM_TOTAL]
    offs = jnp.dot(l200_ref[...], rowtot.astype(jnp.bfloat16),
                   preferred_element_type=jnp.float32)
    gslot = jnp.where(sel, rowcum + offs - 1.0, -1.0)

    kio = lax.broadcasted_iota(jnp.int32, (1, 128), 1).astype(jnp.float32)
    gsT = jnp.transpose(gslot)
    svT = jnp.transpose(sv)
    colT = lax.broadcasted_iota(
        jnp.int32, (NUM_TOTAL, 1), 0).astype(jnp.float32)
    vacc = jnp.zeros((NUM_TOTAL, 128), jnp.float32)
    facc = jnp.zeros((NUM_TOTAL, 128), jnp.float32)
    for q in range(Q):
        eq = gsT[:, q:q + 1] == kio
        vacc = vacc + jnp.where(eq, svT[:, q:q + 1], 0.0)
        facc = facc + jnp.where(eq, q * NUM_CLASSES + colT, 0.0)
    selv = jnp.sum(vacc, axis=0, keepdims=True)
    selfi = jnp.sum(facc, axis=0, keepdims=True)

    vrow_b = jnp.broadcast_to(selv, (128, 128))
    firow_b = jnp.broadcast_to(selfi, (128, 128))
    vcol_b = jnp.broadcast_to(jnp.transpose(selv), (128, 128))
    ficol_b = jnp.broadcast_to(jnp.transpose(selfi), (128, 128))
    gt = (vrow_b > vcol_b) | ((vrow_b == vcol_b) & (firow_b < ficol_b))
    rankcol = jnp.sum(gt.astype(jnp.float32), axis=1, keepdims=True)
    krow = lax.broadcasted_iota(jnp.int32, (1, 128), 1).astype(jnp.float32)
    hot = (rankcol == krow).astype(jnp.bfloat16)
    svh = selv.astype(jnp.bfloat16)
    sr1 = selv - svh.astype(jnp.float32)
    svm = sr1.astype(jnp.bfloat16)
    svl = (sr1 - svm.astype(jnp.float32)).astype(jnp.bfloat16)
    out_v = (jnp.dot(svh, hot, preferred_element_type=jnp.float32)
             + jnp.dot(svm, hot, preferred_element_type=jnp.float32)
             + jnp.dot(svl, hot, preferred_element_type=jnp.float32))
    sfh = selfi.astype(jnp.bfloat16)
    sfl = (selfi - sfh.astype(jnp.float32)).astype(jnp.bfloat16)
    out_f = (jnp.dot(sfh, hot, preferred_element_type=jnp.float32)
             + jnp.dot(sfl, hot, preferred_element_type=jnp.float32))
    fi_i = out_f.astype(jnp.int32)
    lab_ref[...] = fi_i % NUM_CLASSES
    qid_ref[...] = fi_i // NUM_CLASSES
    ovm_mat = jnp.broadcast_to(jnp.transpose(out_v), (128, 128))
    vals3_ref[...] = ovm_mat.reshape(128, 1, 128)


def _run_topk(mask_cls2d):
    u134 = jnp.asarray(np.triu(np.ones((NUM_TOTAL, NUM_TOTAL))),
                       dtype=jnp.bfloat16)
    l200 = jnp.asarray(np.tril(np.ones((Q, Q)), k=-1), dtype=jnp.bfloat16)
    vals3, lab, qid = pl.pallas_call(
        _topk_kernel,
        out_shape=(
            jax.ShapeDtypeStruct((128, 1, 128), jnp.float32),
            jax.ShapeDtypeStruct((1, 128), jnp.int32),
            jax.ShapeDtypeStruct((1, 128), jnp.int32),
        ),
    )(mask_cls2d, u134, l200)
    return vals3, lab[0, :TOPK], qid[0, :TOPK]


def _split3_bf16(a):
    ah = a.astype(jnp.bfloat16)
    r1 = a - ah.astype(jnp.float32)
    am = r1.astype(jnp.bfloat16)
    al = (r1 - am.astype(jnp.float32)).astype(jnp.bfloat16)
    return ah, am, al


def _mask_kernel(qid_ref, *refs):
    xrefs = refs[:MASKS_PER_STEP]
    u_ref, ut_ref, tv_ref, pred_ref, score_ref = refs[MASKS_PER_STEP:]
    u = u_ref[...]
    ut = ut_ref[...]
    for j, xr in enumerate(xrefs):
        x = xr[0]
        xh, xm, xl = _split3_bf16(x)
        y = (jnp.dot(u, xh, preferred_element_type=jnp.float32)
             + jnp.dot(u, xm, preferred_element_type=jnp.float32)
             + jnp.dot(u, xl, preferred_element_type=jnp.float32))
        yh, ym, yl = _split3_bf16(y)
        z = (jnp.dot(yh, ut, preferred_element_type=jnp.float32)
             + jnp.dot(ym, ut, preferred_element_type=jnp.float32)
             + jnp.dot(yl, ut, preferred_element_type=jnp.float32))
        sig = jax.nn.sigmoid(z)
        pos = z > 0.0
        pred_ref[j] = pos
        posf = pos.astype(jnp.float32)
        num = jnp.sum(sig * posf)
        den = jnp.sum(posf)
        score_ref[j, 0, :] = tv_ref[j, 0, :] * num / (den + 1e-6)


def _bilinear_matrix(out_size, in_size):
    sample = (np.arange(out_size) + 0.5) * (in_size / out_size) - 0.5
    w = np.maximum(0.0, 1.0 - np.abs(sample[:, None] - np.arange(in_size)[None, :]))
    w = w / w.sum(axis=1, keepdims=True)
    return w.astype(np.float32)


def _run_masks(mask_pred3d, qid, tv):
    u_np = _bilinear_matrix(OUT_HW, IN_HW)
    u = jnp.asarray(u_np, dtype=jnp.bfloat16)
    ut = jnp.asarray(u_np.T, dtype=jnp.bfloat16)
    grid_spec = pltpu.PrefetchScalarGridSpec(
        num_scalar_prefetch=1,
        grid=(TOPK // MASKS_PER_STEP,),
        in_specs=(
            [pl.BlockSpec(
                (1, IN_HW, IN_HW),
                functools.partial(
                    lambda jj, i, qid_ref: (qid_ref[MASKS_PER_STEP * i + jj],
                                            0, 0), j))
             for j in range(MASKS_PER_STEP)] +
            [pl.BlockSpec((OUT_HW, IN_HW), lambda i, qid_ref: (0, 0)),
             pl.BlockSpec((IN_HW, OUT_HW), lambda i, qid_ref: (0, 0)),
             pl.BlockSpec((MASKS_PER_STEP, 1, 128),
                          lambda i, qid_ref: (i, 0, 0))]
        ),
        out_specs=[
            pl.BlockSpec((MASKS_PER_STEP, OUT_HW, OUT_HW),
                         lambda i, qid_ref: (i, 0, 0)),
            pl.BlockSpec((MASKS_PER_STEP, 1, 128),
                         lambda i, qid_ref: (i, 0, 0)),
        ],
    )
    pred, score = pl.pallas_call(
        _mask_kernel,
        grid_spec=grid_spec,
        out_shape=(
            jax.ShapeDtypeStruct((TOPK, OUT_HW, OUT_HW), jnp.bool_),
            jax.ShapeDtypeStruct((TOPK, 1, 128), jnp.float32),
        ),
    )(qid, *([mask_pred3d] * MASKS_PER_STEP), u, ut, tv)
    return pred, score[:, 0, 0]


def kernel(mask_cls, mask_pred):
    mask_cls2d = mask_cls.reshape(Q, NUM_TOTAL)
    vals3, labels, qid = _run_topk(mask_cls2d)
    pred_masks, final_scores = _run_masks(mask_pred[0], qid, vals3)
    return final_scores, labels, pred_masks
